# Initial kernel scaffold; baseline (speedup 1.0000x reference)
import jax
import jax.numpy as jnp
from jax.experimental import pallas as pl


def kernel(virus_data, host_data, coexistence_data, virus_edge_index, host_edge_index, coexistence_edge_index, coexistence_edge_index_t, virus_edge_weight, host_edge_weight, W_gat_v, att_src_v, att_dst_v, b_gat_v, W_gat_h, att_src_h, att_dst_h, b_gat_h, W_gat_vh, att_src_vh, att_dst_vh, b_gat_vh, W_gat_hv, att_src_hv, att_dst_hv, b_gat_hv, W_lin_v, b_lin_v, W_lin_h, b_lin_h, bn_gamma, bn_beta, bn_mean, bn_var):
    raise NotImplementedError("write your pallas kernel here")


# speedup vs baseline: 12.8536x; 12.8536x over previous
import functools

import jax
import jax.numpy as jnp
from jax import lax
from jax.experimental import pallas as pl
from jax.experimental.pallas import tpu as pltpu
from jax.experimental.pallas import tpu_sc as plsc

_H = 3
_D = 128
_NBIG = 4096
_NCO = 512
_NT = _NBIG + _NCO
_EBIG = 262144 + _NBIG
_ECO = 16384 + _NCO
_ECORE = _EBIG + _ECO
_NSUB = 16
_EPT = _ECORE // _NSUB
_NCHUNK = _EPT // 16
_DEN = _NT * _H
_RSEG = _DEN // _NSUB


def _prep_a_body(vd, hd, co, cot, wv, wh, wvh, whv, av, ah, avhv, avhh):
    av[...] = jnp.dot(vd[...], wv[...], preferred_element_type=jnp.float32)
    ah[...] = jnp.dot(hd[...], wh[...], preferred_element_type=jnp.float32)
    avhv[...] = jnp.dot(co[...], wvh[...], preferred_element_type=jnp.float32)
    avhh[...] = jnp.dot(cot[...], whv[...], preferred_element_type=jnp.float32)


def _prep_a(vd, hd, co, cot, wv, wh, wvh, whv):
    n, nc = vd.shape[0], co.shape[0]
    return pl.pallas_call(
        _prep_a_body,
        out_shape=[
            jax.ShapeDtypeStruct((n, 2 * _H), jnp.float32),
            jax.ShapeDtypeStruct((n, 2 * _H), jnp.float32),
            jax.ShapeDtypeStruct((nc, 2 * _H), jnp.float32),
            jax.ShapeDtypeStruct((nc, 2 * _H), jnp.float32),
        ],
    )(vd, hd, co, cot, wv, wh, wvh, whv)


def _prep_h_body(vd, hd, wv, wh, sv, sh, vh_out, hh_out):
    t = jnp.dot(vd[...], wv[...], preferred_element_type=jnp.float32) + sv[...]
    vh_out[...] = jnp.where(t >= 0.0, t, 0.01 * t)
    u = jnp.dot(hd[...], wh[...], preferred_element_type=jnp.float32) + sh[...]
    hh_out[...] = jnp.where(u >= 0.0, u, 0.01 * u)


def _prep_h(vd, hd, wv, wh, sv, sh):
    n = vd.shape[0]
    return pl.pallas_call(
        _prep_h_body,
        out_shape=[
            jax.ShapeDtypeStruct((n, _D), jnp.float32),
            jax.ShapeDtypeStruct((n, _D), jnp.float32),
        ],
    )(vd, hd, wv, wh, sv, sh)


def _bigmm_body(vh_ref, hht_ref, p_ref, p2_ref):
    t = jnp.dot(vh_ref[...], hht_ref[...], preferred_element_type=jnp.float32)
    p_ref[...] = t
    p2_ref[...] = t + t


def _bigmm(vh, hht):
    n = vh.shape[0]
    m = hht.shape[1]
    tm = 256
    return pl.pallas_call(
        _bigmm_body,
        grid=(n // tm,),
        in_specs=[
            pl.BlockSpec((tm, _D), lambda i: (i, 0)),
            pl.BlockSpec((_D, m), lambda i: (0, 0)),
        ],
        out_specs=[
            pl.BlockSpec((tm, m), lambda i: (i, 0)),
            pl.BlockSpec((tm, m), lambda i: (i, 0)),
        ],
        out_shape=[
            jax.ShapeDtypeStruct((n, m), jnp.float32),
            jax.ShapeDtypeStruct((n, m), jnp.float32),
        ],
    )(vh, hht)


@functools.cache
def _make_alpha_kernel():
    mesh = plsc.VectorSubcoreMesh(core_axis_name="c", subcore_axis_name="s")
    return functools.partial(
        pl.kernel,
        mesh=mesh,
        compiler_params=pltpu.CompilerParams(needs_layout_passes=False),
        out_type=jax.ShapeDtypeStruct((2 * _ECORE * _H,), jnp.float32),
        scratch_types=[
            pltpu.VMEM((_EPT,), jnp.int32),
            pltpu.VMEM((_EPT,), jnp.int32),
            pltpu.VMEM((_NT * 2 * _H,), jnp.float32),
            pltpu.VMEM((_DEN,), jnp.float32),
            pltpu.VMEM((_EPT * _H // 2,), jnp.float32),
            pltpu.VMEM((_RSEG,), jnp.float32),
            pltpu.VMEM_SHARED((_NSUB * _DEN,), jnp.float32),
            pltpu.VMEM_SHARED((_DEN,), jnp.float32),
        ],
    )(_alpha_body)


def _alpha_body(src_hbm, dst_hbm, tab_hbm, out_hbm,
                src_v, dst_v, tab_v, den_v, out3_v, tmp_v, stage, final_sh):
    c = lax.axis_index("c")
    s = lax.axis_index("s")
    ebase = c * _ECORE + s * _EPT
    pltpu.sync_copy(src_hbm.at[pl.ds(ebase, _EPT)], src_v)
    pltpu.sync_copy(dst_hbm.at[pl.ds(ebase, _EPT)], dst_v)
    pltpu.sync_copy(tab_hbm.at[pl.ds(c * _NT * 2 * _H, _NT * 2 * _H)], tab_v)

    zero16 = jnp.zeros((16,), jnp.float32)
    it3 = lax.iota(jnp.int32, 16) * _H

    @plsc.parallel_loop(0, _DEN // 16, unroll=4)
    def _zero(i):
        den_v[pl.ds(i * 16, 16)] = zero16

    @plsc.parallel_loop(0, _NCHUNK, unroll=2)
    def _pass1(i):
        off = pl.multiple_of(i * 16, 16)
        isrc = src_v[pl.ds(off, 16)]
        idst = dst_v[pl.ds(off, 16)]
        i6s = isrc * (2 * _H)
        i6d = idst * (2 * _H) + _H
        i3d = idst * _H
        for h in range(_H):
            a = (plsc.load_gather(tab_v, [i6s + h])
                 + plsc.load_gather(tab_v, [i6d + h]))
            a = jnp.where(a >= 0.0, a, 0.2 * a)
            plsc.addupdate_scatter(den_v, [i3d + h], jnp.exp(a))

    pltpu.sync_copy(den_v, stage.at[pl.ds(s * _DEN, _DEN)])
    plsc.subcore_barrier()

    rbase = s * _RSEG

    def _accrow(r, _):
        @pl.when(r != s)
        def _():
            pltpu.sync_copy(stage.at[pl.ds(r * _DEN + rbase, _RSEG)], tmp_v)

            @plsc.parallel_loop(0, _RSEG // 16, unroll=2)
            def _accvec(j):
                off = pl.multiple_of(j * 16, 16)
                den_v[pl.ds(rbase + off, 16)] = (
                    den_v[pl.ds(rbase + off, 16)] + tmp_v[pl.ds(off, 16)])
        return 0
    lax.fori_loop(0, _NSUB, _accrow, 0)

    @plsc.parallel_loop(0, _RSEG // 16, unroll=2)
    def _recip(j):
        off = pl.multiple_of(j * 16, 16)
        den_v[pl.ds(rbase + off, 16)] = 1.0 / (
            den_v[pl.ds(rbase + off, 16)] + 1e-16)

    pltpu.sync_copy(den_v.at[pl.ds(rbase, _RSEG)],
                    final_sh.at[pl.ds(rbase, _RSEG)])
    plsc.subcore_barrier()
    pltpu.sync_copy(final_sh, den_v)

    half = _NCHUNK // 2
    for r in range(2):
        @plsc.parallel_loop(r * half, (r + 1) * half, unroll=7)
        def _pass2(i):
            off = pl.multiple_of(i * 16, 16)
            isrc = src_v[pl.ds(off, 16)]
            idst = dst_v[pl.ds(off, 16)]
            i6s = isrc * (2 * _H)
            i6d = idst * (2 * _H) + _H
            i3d = idst * _H
            o3 = it3 + _H * off - (r * half * 16 * _H)
            for h in range(_H):
                a = (plsc.load_gather(tab_v, [i6s + h])
                     + plsc.load_gather(tab_v, [i6d + h]))
                a = jnp.where(a >= 0.0, a, 0.2 * a)
                rden = plsc.load_gather(den_v, [i3d + h])
                plsc.store_scatter(out3_v, [o3 + h], jnp.exp(a) * rden)

        pltpu.sync_copy(
            out3_v,
            out_hbm.at[pl.ds(ebase * _H + r * half * 16 * _H,
                             half * 16 * _H)])


def _fold_att(W, att):
    return jnp.einsum("khd,hd->kh", W.reshape(W.shape[0], _H, _D), att[0])


def kernel(virus_data, host_data, coexistence_data, virus_edge_index,
           host_edge_index, coexistence_edge_index, coexistence_edge_index_t,
           virus_edge_weight, host_edge_weight,
           W_gat_v, att_src_v, att_dst_v, b_gat_v,
           W_gat_h, att_src_h, att_dst_h, b_gat_h,
           W_gat_vh, att_src_vh, att_dst_vh, b_gat_vh,
           W_gat_hv, att_src_hv, att_dst_hv, b_gat_hv,
           W_lin_v, b_lin_v, W_lin_h, b_lin_h,
           bn_gamma, bn_beta, bn_mean, bn_var):
    scale = bn_gamma / jnp.sqrt(bn_var + 1e-5)

    wab_v = jnp.concatenate(
        [_fold_att(W_gat_v, att_src_v), _fold_att(W_gat_v, att_dst_v)], axis=1)
    wab_h = jnp.concatenate(
        [_fold_att(W_gat_h, att_src_h), _fold_att(W_gat_h, att_dst_h)], axis=1)
    wab_vh = jnp.concatenate(
        [_fold_att(W_gat_vh, att_src_vh), _fold_att(W_gat_vh, att_dst_vh)],
        axis=1)
    wab_hv = jnp.concatenate(
        [_fold_att(W_gat_hv, att_src_hv), _fold_att(W_gat_hv, att_dst_hv)],
        axis=1)
    sv = (scale * (b_lin_v - bn_mean) + bn_beta)[None, :]
    sh = (scale * (b_lin_h - bn_mean) + bn_beta)[None, :]

    co_t = coexistence_data.T
    a_v, a_h, a_vhv, a_vhh = _prep_a(
        virus_data, host_data, coexistence_data, co_t,
        wab_v, wab_h, wab_vh, wab_hv)

    ar_big = jnp.arange(_NBIG, dtype=jnp.int32)
    ar_co = jnp.arange(_NCO, dtype=jnp.int32)
    sv_full = jnp.concatenate([virus_edge_index[0], ar_big])
    dv_full = jnp.concatenate([virus_edge_index[1], ar_big])
    sh_full = jnp.concatenate([host_edge_index[0], ar_big])
    dh_full = jnp.concatenate([host_edge_index[1], ar_big])
    svhv = jnp.concatenate([coexistence_edge_index_t[0], ar_co])
    dvhv = jnp.concatenate([coexistence_edge_index_t[1], ar_co])
    svhh = jnp.concatenate([coexistence_edge_index[0], ar_co])
    dvhh = jnp.concatenate([coexistence_edge_index[1], ar_co])

    src_all = jnp.concatenate([sv_full, svhv + _NBIG,
                               sh_full, svhh + _NBIG])
    dst_all = jnp.concatenate([dv_full, dvhv + _NBIG,
                               dh_full, dvhh + _NBIG])

    tab_all = jnp.concatenate([a_v, a_vhv, a_h, a_vhh], axis=0).reshape(-1)

    alpha_flat = _make_alpha_kernel()(src_all, dst_all, tab_all)
    alpha2 = alpha_flat.reshape(2 * _ECORE, _H)

    alpha_v = alpha2[:_EBIG]
    alpha_vhv = alpha2[_EBIG:_ECORE]
    alpha_h = alpha2[_ECORE:_ECORE + _EBIG]
    alpha_vhh = alpha2[_ECORE + _EBIG:]

    vh, hh = _prep_h(virus_data, host_data,
                     W_lin_v * scale[None, :], W_lin_h * scale[None, :],
                     sv, sh)
    P, P2 = _bigmm(vh, hh.T)

    ei_v = jnp.stack([sv_full, dv_full])
    ei_h = jnp.stack([sh_full, dh_full])
    ei_vhv = jnp.stack([svhv, dvhv])
    ei_vhh = jnp.stack([svhh, dvhh])

    return (P, P, P2, (ei_v, alpha_v), (ei_h, alpha_h),
            (ei_vhv, alpha_vhv), (ei_vhh, alpha_vhh))


# speedup vs baseline: 54.2980x; 4.2244x over previous
import functools

import jax
import jax.numpy as jnp
from jax import lax
from jax.experimental import pallas as pl
from jax.experimental.pallas import tpu as pltpu
from jax.experimental.pallas import tpu_sc as plsc

_H = 3
_D = 128
_NBIG = 4096
_NCO = 512
_NT = _NBIG + _NCO
_EBIG = 262144 + _NBIG
_ECO = 16384 + _NCO
_ECORE = _EBIG + _ECO
_NSUB = 16
_EPT = _ECORE // _NSUB
_NCHUNK = _EPT // 16
_DEN = _NT * _H
_RSEG = _DEN // _NSUB


def _prep_a_body(vd, hd, co, cot, wv, wh, wvh, whv, av, ah, avhv, avhh):
    av[...] = jnp.dot(vd[...], wv[...], preferred_element_type=jnp.float32)
    ah[...] = jnp.dot(hd[...], wh[...], preferred_element_type=jnp.float32)
    avhv[...] = jnp.dot(co[...], wvh[...], preferred_element_type=jnp.float32)
    avhh[...] = jnp.dot(cot[...], whv[...], preferred_element_type=jnp.float32)


def _prep_a(vd, hd, co, cot, wv, wh, wvh, whv):
    n, nc = vd.shape[0], co.shape[0]
    return pl.pallas_call(
        _prep_a_body,
        out_shape=[
            jax.ShapeDtypeStruct((n, 2 * _H), jnp.float32),
            jax.ShapeDtypeStruct((n, 2 * _H), jnp.float32),
            jax.ShapeDtypeStruct((nc, 2 * _H), jnp.float32),
            jax.ShapeDtypeStruct((nc, 2 * _H), jnp.float32),
        ],
    )(vd, hd, co, cot, wv, wh, wvh, whv)


def _prep_h_body(vd, hd, wv, wh, sv, sh, vh_out, hh_out):
    t = jnp.dot(vd[...], wv[...], preferred_element_type=jnp.float32) + sv[...]
    vh_out[...] = jnp.where(t >= 0.0, t, 0.01 * t)
    u = jnp.dot(hd[...], wh[...], preferred_element_type=jnp.float32) + sh[...]
    hh_out[...] = jnp.where(u >= 0.0, u, 0.01 * u)


def _prep_h(vd, hd, wv, wh, sv, sh):
    n = vd.shape[0]
    return pl.pallas_call(
        _prep_h_body,
        out_shape=[
            jax.ShapeDtypeStruct((n, _D), jnp.float32),
            jax.ShapeDtypeStruct((n, _D), jnp.float32),
        ],
    )(vd, hd, wv, wh, sv, sh)


def _bigmm_body(vh_ref, hht_ref, p_ref, p2_ref):
    t = jnp.dot(vh_ref[...], hht_ref[...], preferred_element_type=jnp.float32)
    p_ref[...] = t
    p2_ref[...] = t + t


def _bigmm(vh, hht):
    n = vh.shape[0]
    m = hht.shape[1]
    tm = 256
    return pl.pallas_call(
        _bigmm_body,
        grid=(n // tm,),
        in_specs=[
            pl.BlockSpec((tm, _D), lambda i: (i, 0)),
            pl.BlockSpec((_D, m), lambda i: (0, 0)),
        ],
        out_specs=[
            pl.BlockSpec((tm, m), lambda i: (i, 0)),
            pl.BlockSpec((tm, m), lambda i: (i, 0)),
        ],
        out_shape=[
            jax.ShapeDtypeStruct((n, m), jnp.float32),
            jax.ShapeDtypeStruct((n, m), jnp.float32),
        ],
    )(vh, hht)


@functools.cache
def _make_alpha_kernel():
    mesh = plsc.VectorSubcoreMesh(core_axis_name="c", subcore_axis_name="s")
    return functools.partial(
        pl.kernel,
        mesh=mesh,
        compiler_params=pltpu.CompilerParams(needs_layout_passes=False),
        out_type=jax.ShapeDtypeStruct((2 * _ECORE * _H,), jnp.float32),
        scratch_types=[
            pltpu.VMEM((_EPT,), jnp.int32),
            pltpu.VMEM((_EPT,), jnp.int32),
            pltpu.VMEM((_NT * 2 * _H,), jnp.float32),
            pltpu.VMEM((_DEN,), jnp.float32),
            pltpu.VMEM((_EPT,), jnp.float32),
            pltpu.VMEM((_RSEG,), jnp.float32),
            pltpu.VMEM_SHARED((_NSUB * _DEN,), jnp.float32),
            pltpu.VMEM_SHARED((_DEN,), jnp.float32),
        ],
    )(_alpha_body)


def _alpha_body(src_hbm, dst_hbm, tab_hbm, out_hbm,
                src_v, dst_v, tab_v, den_v, out3_v, tmp_v, stage, final_sh):
    c = lax.axis_index("c")
    s = lax.axis_index("s")
    ebase = c * _ECORE + s * _EPT
    pltpu.sync_copy(src_hbm.at[pl.ds(ebase, _EPT)], src_v)
    pltpu.sync_copy(dst_hbm.at[pl.ds(ebase, _EPT)], dst_v)
    pltpu.sync_copy(tab_hbm.at[pl.ds(c * _NT * 2 * _H, _NT * 2 * _H)], tab_v)

    zero16 = jnp.zeros((16,), jnp.float32)

    @plsc.parallel_loop(0, _DEN // 16, unroll=4)
    def _zero(i):
        den_v[pl.ds(i * 16, 16)] = zero16

    @plsc.parallel_loop(0, _NCHUNK, unroll=2)
    def _pass1(i):
        off = pl.multiple_of(i * 16, 16)
        isrc = src_v[pl.ds(off, 16)]
        idst = dst_v[pl.ds(off, 16)]
        i6s = isrc * (2 * _H)
        i6d = idst * (2 * _H) + _H
        i3d = idst * _H
        for h in range(_H):
            a = (plsc.load_gather(tab_v, [i6s + h])
                 + plsc.load_gather(tab_v, [i6d + h]))
            a = jnp.where(a >= 0.0, a, 0.2 * a)
            plsc.addupdate_scatter(den_v, [i3d + h], jnp.exp(a))

    pltpu.sync_copy(den_v, stage.at[pl.ds(s * _DEN, _DEN)])
    plsc.subcore_barrier()

    rbase = s * _RSEG

    def _accrow(r, _):
        @pl.when(r != s)
        def _():
            pltpu.sync_copy(stage.at[pl.ds(r * _DEN + rbase, _RSEG)], tmp_v)

            @plsc.parallel_loop(0, _RSEG // 16, unroll=2)
            def _accvec(j):
                off = pl.multiple_of(j * 16, 16)
                den_v[pl.ds(rbase + off, 16)] = (
                    den_v[pl.ds(rbase + off, 16)] + tmp_v[pl.ds(off, 16)])
        return 0
    lax.fori_loop(0, _NSUB, _accrow, 0)

    @plsc.parallel_loop(0, _RSEG // 16, unroll=2)
    def _recip(j):
        off = pl.multiple_of(j * 16, 16)
        den_v[pl.ds(rbase + off, 16)] = 1.0 / (
            den_v[pl.ds(rbase + off, 16)] + 1e-16)

    pltpu.sync_copy(den_v.at[pl.ds(rbase, _RSEG)],
                    final_sh.at[pl.ds(rbase, _RSEG)])
    plsc.subcore_barrier()
    pltpu.sync_copy(final_sh, den_v)

    for h in range(_H):
        @plsc.parallel_loop(0, _NCHUNK, unroll=2)
        def _pass2(i):
            off = pl.multiple_of(i * 16, 16)
            isrc = src_v[pl.ds(off, 16)]
            idst = dst_v[pl.ds(off, 16)]
            a = (plsc.load_gather(tab_v, [isrc * (2 * _H) + h])
                 + plsc.load_gather(tab_v, [idst * (2 * _H) + (_H + h)]))
            a = jnp.where(a >= 0.0, a, 0.2 * a)
            rden = plsc.load_gather(den_v, [idst * _H + h])
            out3_v[pl.ds(off, 16)] = jnp.exp(a) * rden

        pltpu.sync_copy(
            out3_v,
            out_hbm.at[pl.ds((c * _H + h) * _ECORE + s * _EPT, _EPT)])


def _fold_att(W, att):
    return jnp.einsum("khd,hd->kh", W.reshape(W.shape[0], _H, _D), att[0])


def kernel(virus_data, host_data, coexistence_data, virus_edge_index,
           host_edge_index, coexistence_edge_index, coexistence_edge_index_t,
           virus_edge_weight, host_edge_weight,
           W_gat_v, att_src_v, att_dst_v, b_gat_v,
           W_gat_h, att_src_h, att_dst_h, b_gat_h,
           W_gat_vh, att_src_vh, att_dst_vh, b_gat_vh,
           W_gat_hv, att_src_hv, att_dst_hv, b_gat_hv,
           W_lin_v, b_lin_v, W_lin_h, b_lin_h,
           bn_gamma, bn_beta, bn_mean, bn_var):
    scale = bn_gamma / jnp.sqrt(bn_var + 1e-5)

    wab_v = jnp.concatenate(
        [_fold_att(W_gat_v, att_src_v), _fold_att(W_gat_v, att_dst_v)], axis=1)
    wab_h = jnp.concatenate(
        [_fold_att(W_gat_h, att_src_h), _fold_att(W_gat_h, att_dst_h)], axis=1)
    wab_vh = jnp.concatenate(
        [_fold_att(W_gat_vh, att_src_vh), _fold_att(W_gat_vh, att_dst_vh)],
        axis=1)
    wab_hv = jnp.concatenate(
        [_fold_att(W_gat_hv, att_src_hv), _fold_att(W_gat_hv, att_dst_hv)],
        axis=1)
    sv = (scale * (b_lin_v - bn_mean) + bn_beta)[None, :]
    sh = (scale * (b_lin_h - bn_mean) + bn_beta)[None, :]

    co_t = coexistence_data.T
    a_v, a_h, a_vhv, a_vhh = _prep_a(
        virus_data, host_data, coexistence_data, co_t,
        wab_v, wab_h, wab_vh, wab_hv)

    ar_big = jnp.arange(_NBIG, dtype=jnp.int32)
    ar_co = jnp.arange(_NCO, dtype=jnp.int32)
    sv_full = jnp.concatenate([virus_edge_index[0], ar_big])
    dv_full = jnp.concatenate([virus_edge_index[1], ar_big])
    sh_full = jnp.concatenate([host_edge_index[0], ar_big])
    dh_full = jnp.concatenate([host_edge_index[1], ar_big])
    svhv = jnp.concatenate([coexistence_edge_index_t[0], ar_co])
    dvhv = jnp.concatenate([coexistence_edge_index_t[1], ar_co])
    svhh = jnp.concatenate([coexistence_edge_index[0], ar_co])
    dvhh = jnp.concatenate([coexistence_edge_index[1], ar_co])

    src_all = jnp.concatenate([sv_full, svhv + _NBIG,
                               sh_full, svhh + _NBIG])
    dst_all = jnp.concatenate([dv_full, dvhv + _NBIG,
                               dh_full, dvhh + _NBIG])

    tab_all = jnp.concatenate([a_v, a_vhv, a_h, a_vhh], axis=0).reshape(-1)

    alpha_flat = _make_alpha_kernel()(src_all, dst_all, tab_all)
    alpha_all = alpha_flat.reshape(2, _H, _ECORE)

    alpha_v = alpha_all[0, :, :_EBIG].T
    alpha_vhv = alpha_all[0, :, _EBIG:].T
    alpha_h = alpha_all[1, :, :_EBIG].T
    alpha_vhh = alpha_all[1, :, _EBIG:].T

    vh, hh = _prep_h(virus_data, host_data,
                     W_lin_v * scale[None, :], W_lin_h * scale[None, :],
                     sv, sh)
    P, P2 = _bigmm(vh, hh.T)

    ei_v = jnp.stack([sv_full, dv_full])
    ei_h = jnp.stack([sh_full, dh_full])
    ei_vhv = jnp.stack([svhv, dvhv])
    ei_vhh = jnp.stack([svhh, dvhh])

    return (P, P, P2, (ei_v, alpha_v), (ei_h, alpha_h),
            (ei_vhv, alpha_vhv), (ei_vhh, alpha_vhh))


# speedup vs baseline: 66.6663x; 1.2278x over previous
import functools

import jax
import jax.numpy as jnp
from jax import lax
from jax.experimental import pallas as pl
from jax.experimental.pallas import tpu as pltpu
from jax.experimental.pallas import tpu_sc as plsc

_H = 3
_D = 128
_NBIG = 4096
_NCO = 512
_NT = _NBIG + _NCO
_EBIG = 262144 + _NBIG
_ECO = 16384 + _NCO
_ECORE = _EBIG + _ECO
_NSUB = 16
_EPT = _ECORE // _NSUB
_NCHUNK = _EPT // 16
_DEN = _NT * _H
_RSEG = _DEN // _NSUB


def _prep_a_body(vd, hd, co, cot, wv, wh, wvh, whv, av, ah, avhv, avhh):
    av[...] = jnp.dot(vd[...], wv[...], preferred_element_type=jnp.float32)
    ah[...] = jnp.dot(hd[...], wh[...], preferred_element_type=jnp.float32)
    avhv[...] = jnp.dot(co[...], wvh[...], preferred_element_type=jnp.float32)
    avhh[...] = jnp.dot(cot[...], whv[...], preferred_element_type=jnp.float32)


def _prep_a(vd, hd, co, cot, wv, wh, wvh, whv):
    n, nc = vd.shape[0], co.shape[0]
    return pl.pallas_call(
        _prep_a_body,
        out_shape=[
            jax.ShapeDtypeStruct((n, 2 * _H), jnp.float32),
            jax.ShapeDtypeStruct((n, 2 * _H), jnp.float32),
            jax.ShapeDtypeStruct((nc, 2 * _H), jnp.float32),
            jax.ShapeDtypeStruct((nc, 2 * _H), jnp.float32),
        ],
    )(vd, hd, co, cot, wv, wh, wvh, whv)


def _prep_h_body(vd, hd, wv, wh, sc, sv, sh, vh_out, hht_out):
    wvs = wv[...] * sc[...]
    whs = wh[...] * sc[...]
    t = jnp.dot(vd[...], wvs, preferred_element_type=jnp.float32) + sv[...]
    vh_out[...] = jnp.where(t >= 0.0, t, 0.01 * t)
    u = jnp.dot(hd[...], whs, preferred_element_type=jnp.float32) + sh[...]
    hht_out[...] = jnp.where(u >= 0.0, u, 0.01 * u).T


def _prep_h(vd, hd, wv, wh, sc, sv, sh):
    n = vd.shape[0]
    return pl.pallas_call(
        _prep_h_body,
        out_shape=[
            jax.ShapeDtypeStruct((n, _D), jnp.float32),
            jax.ShapeDtypeStruct((_D, n), jnp.float32),
        ],
    )(vd, hd, wv, wh, sc, sv, sh)


def _edges_body(vei, hei, coei, cotei, arb, arc,
                eiv, eih, eivhv, eivhh, src_all, dst_all):
    arb_v = arb[...]
    arc_v = arc[...]
    arc_off = arc_v + _NBIG
    for row in (0, 1):
        vr = vei[row, :]
        hr = hei[row, :]
        cor = coei[row, :]
        cotr = cotei[row, :]
        eiv[row, pl.ds(0, _EBIG - _NBIG)] = vr
        eiv[row, pl.ds(_EBIG - _NBIG, _NBIG)] = arb_v
        eih[row, pl.ds(0, _EBIG - _NBIG)] = hr
        eih[row, pl.ds(_EBIG - _NBIG, _NBIG)] = arb_v
        eivhv[row, pl.ds(0, _ECO - _NCO)] = cotr
        eivhv[row, pl.ds(_ECO - _NCO, _NCO)] = arc_v
        eivhh[row, pl.ds(0, _ECO - _NCO)] = cor
        eivhh[row, pl.ds(_ECO - _NCO, _NCO)] = arc_v
        out = src_all if row == 0 else dst_all
        out[pl.ds(0, _EBIG - _NBIG)] = vr
        out[pl.ds(_EBIG - _NBIG, _NBIG)] = arb_v
        out[pl.ds(_EBIG, _ECO - _NCO)] = cotr + _NBIG
        out[pl.ds(_EBIG + _ECO - _NCO, _NCO)] = arc_off
        out[pl.ds(_ECORE, _EBIG - _NBIG)] = hr
        out[pl.ds(_ECORE + _EBIG - _NBIG, _NBIG)] = arb_v
        out[pl.ds(_ECORE + _EBIG, _ECO - _NCO)] = cor + _NBIG
        out[pl.ds(_ECORE + _EBIG + _ECO - _NCO, _NCO)] = arc_off


def _edges(vei, hei, coei, cotei, arb, arc):
    return pl.pallas_call(
        _edges_body,
        out_shape=[
            jax.ShapeDtypeStruct((2, _EBIG), jnp.int32),
            jax.ShapeDtypeStruct((2, _EBIG), jnp.int32),
            jax.ShapeDtypeStruct((2, _ECO), jnp.int32),
            jax.ShapeDtypeStruct((2, _ECO), jnp.int32),
            jax.ShapeDtypeStruct((2 * _ECORE,), jnp.int32),
            jax.ShapeDtypeStruct((2 * _ECORE,), jnp.int32),
        ],
    )(vei, hei, coei, cotei, arb, arc)


def _bigmm_body(vh_ref, hht_ref, pa_ref, pb_ref, p2_ref):
    t = jnp.dot(vh_ref[...], hht_ref[...], preferred_element_type=jnp.float32)
    pa_ref[...] = t
    pb_ref[...] = t
    p2_ref[...] = t + t


def _bigmm(vh, hht):
    n = vh.shape[0]
    m = hht.shape[1]
    tm = 256
    return pl.pallas_call(
        _bigmm_body,
        grid=(n // tm,),
        in_specs=[
            pl.BlockSpec((tm, _D), lambda i: (i, 0)),
            pl.BlockSpec((_D, m), lambda i: (0, 0)),
        ],
        out_specs=[
            pl.BlockSpec((tm, m), lambda i: (i, 0)),
            pl.BlockSpec((tm, m), lambda i: (i, 0)),
            pl.BlockSpec((tm, m), lambda i: (i, 0)),
        ],
        out_shape=[
            jax.ShapeDtypeStruct((n, m), jnp.float32),
            jax.ShapeDtypeStruct((n, m), jnp.float32),
            jax.ShapeDtypeStruct((n, m), jnp.float32),
        ],
    )(vh, hht)


@functools.cache
def _make_alpha_kernel():
    mesh = plsc.VectorSubcoreMesh(core_axis_name="c", subcore_axis_name="s")
    return functools.partial(
        pl.kernel,
        mesh=mesh,
        compiler_params=pltpu.CompilerParams(needs_layout_passes=False),
        out_type=jax.ShapeDtypeStruct((2 * _ECORE * _H,), jnp.float32),
        scratch_types=[
            pltpu.VMEM((_EPT,), jnp.int32),
            pltpu.VMEM((_EPT,), jnp.int32),
            pltpu.VMEM((_NT * 2 * _H,), jnp.float32),
            pltpu.VMEM((_DEN,), jnp.float32),
            pltpu.VMEM((_EPT,), jnp.float32),
            pltpu.VMEM((_RSEG,), jnp.float32),
            pltpu.VMEM_SHARED((_NSUB * _DEN,), jnp.float32),
            pltpu.VMEM_SHARED((_DEN,), jnp.float32),
        ],
    )(_alpha_body)


def _alpha_body(src_hbm, dst_hbm, tab_hbm, out_hbm,
                src_v, dst_v, tab_v, den_v, out3_v, tmp_v, stage, final_sh):
    c = lax.axis_index("c")
    s = lax.axis_index("s")
    ebase = c * _ECORE + s * _EPT
    pltpu.sync_copy(src_hbm.at[pl.ds(ebase, _EPT)], src_v)
    pltpu.sync_copy(dst_hbm.at[pl.ds(ebase, _EPT)], dst_v)
    pltpu.sync_copy(tab_hbm.at[pl.ds(c * _NT * 2 * _H, _NT * 2 * _H)], tab_v)

    zero16 = jnp.zeros((16,), jnp.float32)

    @plsc.parallel_loop(0, _DEN // 16, unroll=4)
    def _zero(i):
        den_v[pl.ds(i * 16, 16)] = zero16

    @plsc.parallel_loop(0, _NCHUNK, unroll=2)
    def _pass1(i):
        off = pl.multiple_of(i * 16, 16)
        isrc = src_v[pl.ds(off, 16)]
        idst = dst_v[pl.ds(off, 16)]
        i6s = isrc * (2 * _H)
        i6d = idst * (2 * _H) + _H
        i3d = idst * _H
        for h in range(_H):
            a = (plsc.load_gather(tab_v, [i6s + h])
                 + plsc.load_gather(tab_v, [i6d + h]))
            a = jnp.where(a >= 0.0, a, 0.2 * a)
            plsc.addupdate_scatter(den_v, [i3d + h], jnp.exp(a))

    pltpu.sync_copy(den_v, stage.at[pl.ds(s * _DEN, _DEN)])
    plsc.subcore_barrier()

    rbase = s * _RSEG

    def _accrow(r, _):
        @pl.when(r != s)
        def _():
            pltpu.sync_copy(stage.at[pl.ds(r * _DEN + rbase, _RSEG)], tmp_v)

            @plsc.parallel_loop(0, _RSEG // 16, unroll=2)
            def _accvec(j):
                off = pl.multiple_of(j * 16, 16)
                den_v[pl.ds(rbase + off, 16)] = (
                    den_v[pl.ds(rbase + off, 16)] + tmp_v[pl.ds(off, 16)])
        return 0
    lax.fori_loop(0, _NSUB, _accrow, 0)

    @plsc.parallel_loop(0, _RSEG // 16, unroll=2)
    def _recip(j):
        off = pl.multiple_of(j * 16, 16)
        den_v[pl.ds(rbase + off, 16)] = 1.0 / (
            den_v[pl.ds(rbase + off, 16)] + 1e-16)

    pltpu.sync_copy(den_v.at[pl.ds(rbase, _RSEG)],
                    final_sh.at[pl.ds(rbase, _RSEG)])
    plsc.subcore_barrier()
    pltpu.sync_copy(final_sh, den_v)

    for h in range(_H):
        @plsc.parallel_loop(0, _NCHUNK, unroll=2)
        def _pass2(i):
            off = pl.multiple_of(i * 16, 16)
            isrc = src_v[pl.ds(off, 16)]
            idst = dst_v[pl.ds(off, 16)]
            a = (plsc.load_gather(tab_v, [isrc * (2 * _H) + h])
                 + plsc.load_gather(tab_v, [idst * (2 * _H) + (_H + h)]))
            a = jnp.where(a >= 0.0, a, 0.2 * a)
            rden = plsc.load_gather(den_v, [idst * _H + h])
            out3_v[pl.ds(off, 16)] = jnp.exp(a) * rden

        pltpu.sync_copy(
            out3_v,
            out_hbm.at[pl.ds((c * _H + h) * _ECORE + s * _EPT, _EPT)])


def _fold_att(W, att):
    return jnp.einsum("khd,hd->kh", W.reshape(W.shape[0], _H, _D), att[0])


def kernel(virus_data, host_data, coexistence_data, virus_edge_index,
           host_edge_index, coexistence_edge_index, coexistence_edge_index_t,
           virus_edge_weight, host_edge_weight,
           W_gat_v, att_src_v, att_dst_v, b_gat_v,
           W_gat_h, att_src_h, att_dst_h, b_gat_h,
           W_gat_vh, att_src_vh, att_dst_vh, b_gat_vh,
           W_gat_hv, att_src_hv, att_dst_hv, b_gat_hv,
           W_lin_v, b_lin_v, W_lin_h, b_lin_h,
           bn_gamma, bn_beta, bn_mean, bn_var):
    scale = bn_gamma / jnp.sqrt(bn_var + 1e-5)

    wab_v = jnp.concatenate(
        [_fold_att(W_gat_v, att_src_v), _fold_att(W_gat_v, att_dst_v)], axis=1)
    wab_h = jnp.concatenate(
        [_fold_att(W_gat_h, att_src_h), _fold_att(W_gat_h, att_dst_h)], axis=1)
    wab_vh = jnp.concatenate(
        [_fold_att(W_gat_vh, att_src_vh), _fold_att(W_gat_vh, att_dst_vh)],
        axis=1)
    wab_hv = jnp.concatenate(
        [_fold_att(W_gat_hv, att_src_hv), _fold_att(W_gat_hv, att_dst_hv)],
        axis=1)
    sv = (scale * (b_lin_v - bn_mean) + bn_beta)[None, :]
    sh = (scale * (b_lin_h - bn_mean) + bn_beta)[None, :]

    co_t = coexistence_data.T
    a_v, a_h, a_vhv, a_vhh = _prep_a(
        virus_data, host_data, coexistence_data, co_t,
        wab_v, wab_h, wab_vh, wab_hv)

    ar_big = jnp.arange(_NBIG, dtype=jnp.int32)
    ar_co = jnp.arange(_NCO, dtype=jnp.int32)
    ei_v, ei_h, ei_vhv, ei_vhh, src_all, dst_all = _edges(
        virus_edge_index, host_edge_index, coexistence_edge_index,
        coexistence_edge_index_t, ar_big, ar_co)

    tab_all = jnp.concatenate([a_v, a_vhv, a_h, a_vhh], axis=0).reshape(-1)

    alpha_flat = _make_alpha_kernel()(src_all, dst_all, tab_all)
    alpha_all = alpha_flat.reshape(2, _H, _ECORE)

    alpha_v = alpha_all[0, :, :_EBIG].T
    alpha_vhv = alpha_all[0, :, _EBIG:].T
    alpha_h = alpha_all[1, :, :_EBIG].T
    alpha_vhh = alpha_all[1, :, _EBIG:].T

    vh, hht = _prep_h(virus_data, host_data, W_lin_v, W_lin_h,
                      scale[None, :], sv, sh)
    P, Pb, P2 = _bigmm(vh, hht)

    return (P, Pb, P2, (ei_v, alpha_v), (ei_h, alpha_h),
            (ei_vhv, alpha_vhv), (ei_vhh, alpha_vhh))


# speedup vs baseline: 75.1731x; 1.1276x over previous
import functools

import jax
import jax.numpy as jnp
from jax import lax
from jax.experimental import pallas as pl
from jax.experimental.pallas import tpu as pltpu
from jax.experimental.pallas import tpu_sc as plsc

_H = 3
_D = 128
_NBIG = 4096
_NCO = 512
_NT = _NBIG + _NCO
_EBIG = 262144 + _NBIG
_ECO = 16384 + _NCO
_ECORE = _EBIG + _ECO
_NSUB = 16
_EPT = _ECORE // _NSUB
_NCHUNK = _EPT // 16
_DEN = _NT * _H
_RSEG = _DEN // _NSUB


def _prep_body(vd, hd, co, cot, wgv, av_m, wgh, ah_m, wgvh, avh_m,
               wghv, ahv_m, wlv, wlh, sc, sv, sh, vh_out, hht_out, tabt):
    def logits(x, wg, a_m, col):
        wab = jnp.dot(wg[...], a_m[...], preferred_element_type=jnp.float32)
        t = jnp.dot(x[...], wab, preferred_element_type=jnp.float32)
        tabt[:, pl.ds(col, t.shape[0])] = t.T

    logits(vd, wgv, av_m, 0)
    logits(co, wgvh, avh_m, _NBIG)
    logits(hd, wgh, ah_m, _NT)
    logits(cot, wghv, ahv_m, _NT + _NBIG)

    t = (jnp.dot(vd[...], wlv[...] * sc[...],
                 preferred_element_type=jnp.float32) + sv[...])
    vh_out[...] = jnp.where(t >= 0.0, t, 0.01 * t)
    u = (jnp.dot(hd[...], wlh[...] * sc[...],
                 preferred_element_type=jnp.float32) + sh[...])
    hht_out[...] = jnp.where(u >= 0.0, u, 0.01 * u).T


def _prep(vd, hd, co, cot, wgv, av_m, wgh, ah_m, wgvh, avh_m, wghv, ahv_m,
          wlv, wlh, sc, sv, sh):
    n = vd.shape[0]
    return pl.pallas_call(
        _prep_body,
        out_shape=[
            jax.ShapeDtypeStruct((n, _D), jnp.float32),
            jax.ShapeDtypeStruct((_D, n), jnp.float32),
            jax.ShapeDtypeStruct((8, 2 * _NT), jnp.float32),
        ],
    )(vd, hd, co, cot, wgv, av_m, wgh, ah_m, wgvh, avh_m, wghv, ahv_m,
      wlv, wlh, sc, sv, sh)


def _edges_body(vei, hei, coei, cotei, arb, arc,
                eiv, eih, eivhv, eivhh, src_all, dst_all):
    arb_v = arb[...]
    arc_v = arc[...]
    arc_off = arc_v + _NBIG
    for row in (0, 1):
        vr = vei[row, :]
        hr = hei[row, :]
        cor = coei[row, :]
        cotr = cotei[row, :]
        eiv[row, pl.ds(0, _EBIG - _NBIG)] = vr
        eiv[row, pl.ds(_EBIG - _NBIG, _NBIG)] = arb_v
        eih[row, pl.ds(0, _EBIG - _NBIG)] = hr
        eih[row, pl.ds(_EBIG - _NBIG, _NBIG)] = arb_v
        eivhv[row, pl.ds(0, _ECO - _NCO)] = cotr
        eivhv[row, pl.ds(_ECO - _NCO, _NCO)] = arc_v
        eivhh[row, pl.ds(0, _ECO - _NCO)] = cor
        eivhh[row, pl.ds(_ECO - _NCO, _NCO)] = arc_v
        out = src_all if row == 0 else dst_all
        out[pl.ds(0, _EBIG - _NBIG)] = vr
        out[pl.ds(_EBIG - _NBIG, _NBIG)] = arb_v
        out[pl.ds(_EBIG, _ECO - _NCO)] = cotr + _NBIG
        out[pl.ds(_EBIG + _ECO - _NCO, _NCO)] = arc_off
        out[pl.ds(_ECORE, _EBIG - _NBIG)] = hr
        out[pl.ds(_ECORE + _EBIG - _NBIG, _NBIG)] = arb_v
        out[pl.ds(_ECORE + _EBIG, _ECO - _NCO)] = cor + _NBIG
        out[pl.ds(_ECORE + _EBIG + _ECO - _NCO, _NCO)] = arc_off


def _edges(vei, hei, coei, cotei, arb, arc):
    return pl.pallas_call(
        _edges_body,
        out_shape=[
            jax.ShapeDtypeStruct((2, _EBIG), jnp.int32),
            jax.ShapeDtypeStruct((2, _EBIG), jnp.int32),
            jax.ShapeDtypeStruct((2, _ECO), jnp.int32),
            jax.ShapeDtypeStruct((2, _ECO), jnp.int32),
            jax.ShapeDtypeStruct((2 * _ECORE,), jnp.int32),
            jax.ShapeDtypeStruct((2 * _ECORE,), jnp.int32),
        ],
    )(vei, hei, coei, cotei, arb, arc)


def _bigmm_body(vh_ref, hht_ref, pa_ref, pb_ref, p2_ref):
    t = jnp.dot(vh_ref[...], hht_ref[...], preferred_element_type=jnp.float32)
    pa_ref[...] = t
    pb_ref[...] = t
    p2_ref[...] = t + t


def _bigmm(vh, hht):
    n = vh.shape[0]
    m = hht.shape[1]
    tm = 256
    return pl.pallas_call(
        _bigmm_body,
        grid=(n // tm,),
        in_specs=[
            pl.BlockSpec((tm, _D), lambda i: (i, 0)),
            pl.BlockSpec((_D, m), lambda i: (0, 0)),
        ],
        out_specs=[
            pl.BlockSpec((tm, m), lambda i: (i, 0)),
            pl.BlockSpec((tm, m), lambda i: (i, 0)),
            pl.BlockSpec((tm, m), lambda i: (i, 0)),
        ],
        out_shape=[
            jax.ShapeDtypeStruct((n, m), jnp.float32),
            jax.ShapeDtypeStruct((n, m), jnp.float32),
            jax.ShapeDtypeStruct((n, m), jnp.float32),
        ],
    )(vh, hht)


@functools.cache
def _make_alpha_kernel():
    mesh = plsc.VectorSubcoreMesh(core_axis_name="c", subcore_axis_name="s")
    return functools.partial(
        pl.kernel,
        mesh=mesh,
        compiler_params=pltpu.CompilerParams(needs_layout_passes=False),
        out_type=jax.ShapeDtypeStruct((2 * _ECORE * _H,), jnp.float32),
        scratch_types=[
            pltpu.VMEM((_EPT,), jnp.int32),
            pltpu.VMEM((_EPT,), jnp.int32),
            pltpu.VMEM((_NT * 2 * _H,), jnp.float32),
            pltpu.VMEM((_DEN,), jnp.float32),
            pltpu.VMEM((_EPT,), jnp.float32),
            pltpu.VMEM((_RSEG,), jnp.float32),
            pltpu.VMEM_SHARED((_NSUB * _DEN,), jnp.float32),
            pltpu.VMEM_SHARED((_DEN,), jnp.float32),
        ],
    )(_alpha_body)


def _alpha_body(src_hbm, dst_hbm, tab_hbm, out_hbm,
                src_v, dst_v, tab_v, den_v, out3_v, tmp_v, stage, final_sh):
    c = lax.axis_index("c")
    s = lax.axis_index("s")
    ebase = c * _ECORE + s * _EPT
    pltpu.sync_copy(src_hbm.at[pl.ds(ebase, _EPT)], src_v)
    pltpu.sync_copy(dst_hbm.at[pl.ds(ebase, _EPT)], dst_v)
    for h in range(2 * _H):
        pltpu.sync_copy(tab_hbm.at[h, pl.ds(c * _NT, _NT)],
                        tab_v.at[pl.ds(h * _NT, _NT)])

    zero16 = jnp.zeros((16,), jnp.float32)

    @plsc.parallel_loop(0, _DEN // 16, unroll=4)
    def _zero(i):
        den_v[pl.ds(i * 16, 16)] = zero16

    @plsc.parallel_loop(0, _NCHUNK, unroll=2)
    def _pass1(i):
        off = pl.multiple_of(i * 16, 16)
        isrc = src_v[pl.ds(off, 16)]
        idst = dst_v[pl.ds(off, 16)]
        i3d = idst * _H
        for h in range(_H):
            a = (plsc.load_gather(tab_v, [isrc + h * _NT])
                 + plsc.load_gather(tab_v, [idst + (_H + h) * _NT]))
            a = jnp.where(a >= 0.0, a, 0.2 * a)
            plsc.addupdate_scatter(den_v, [i3d + h], jnp.exp(a))

    pltpu.sync_copy(den_v, stage.at[pl.ds(s * _DEN, _DEN)])
    plsc.subcore_barrier()

    rbase = s * _RSEG

    def _accrow(r, _):
        @pl.when(r != s)
        def _():
            pltpu.sync_copy(stage.at[pl.ds(r * _DEN + rbase, _RSEG)], tmp_v)

            @plsc.parallel_loop(0, _RSEG // 16, unroll=2)
            def _accvec(j):
                off = pl.multiple_of(j * 16, 16)
                den_v[pl.ds(rbase + off, 16)] = (
                    den_v[pl.ds(rbase + off, 16)] + tmp_v[pl.ds(off, 16)])
        return 0
    lax.fori_loop(0, _NSUB, _accrow, 0)

    @plsc.parallel_loop(0, _RSEG // 16, unroll=2)
    def _recip(j):
        off = pl.multiple_of(j * 16, 16)
        den_v[pl.ds(rbase + off, 16)] = 1.0 / (
            den_v[pl.ds(rbase + off, 16)] + 1e-16)

    pltpu.sync_copy(den_v.at[pl.ds(rbase, _RSEG)],
                    final_sh.at[pl.ds(rbase, _RSEG)])
    plsc.subcore_barrier()
    pltpu.sync_copy(final_sh, den_v)

    for h in range(_H):
        @plsc.parallel_loop(0, _NCHUNK, unroll=2)
        def _pass2(i):
            off = pl.multiple_of(i * 16, 16)
            isrc = src_v[pl.ds(off, 16)]
            idst = dst_v[pl.ds(off, 16)]
            a = (plsc.load_gather(tab_v, [isrc + h * _NT])
                 + plsc.load_gather(tab_v, [idst + (_H + h) * _NT]))
            a = jnp.where(a >= 0.0, a, 0.2 * a)
            rden = plsc.load_gather(den_v, [idst * _H + h])
            out3_v[pl.ds(off, 16)] = jnp.exp(a) * rden

        pltpu.sync_copy(
            out3_v,
            out_hbm.at[pl.ds((c * _H + h) * _ECORE + s * _EPT, _EPT)])


def _att_mat(att_src, att_dst):
    eye = jnp.eye(_H, dtype=jnp.float32)
    a_s = jnp.einsum("hd,hk->hdk", att_src[0], eye).reshape(_H * _D, _H)
    a_d = jnp.einsum("hd,hk->hdk", att_dst[0], eye).reshape(_H * _D, _H)
    return jnp.pad(jnp.concatenate([a_s, a_d], axis=1), ((0, 0), (0, 2)))


def kernel(virus_data, host_data, coexistence_data, virus_edge_index,
           host_edge_index, coexistence_edge_index, coexistence_edge_index_t,
           virus_edge_weight, host_edge_weight,
           W_gat_v, att_src_v, att_dst_v, b_gat_v,
           W_gat_h, att_src_h, att_dst_h, b_gat_h,
           W_gat_vh, att_src_vh, att_dst_vh, b_gat_vh,
           W_gat_hv, att_src_hv, att_dst_hv, b_gat_hv,
           W_lin_v, b_lin_v, W_lin_h, b_lin_h,
           bn_gamma, bn_beta, bn_mean, bn_var):
    scale = bn_gamma / jnp.sqrt(bn_var + 1e-5)

    am_v = _att_mat(att_src_v, att_dst_v)
    am_h = _att_mat(att_src_h, att_dst_h)
    am_vh = _att_mat(att_src_vh, att_dst_vh)
    am_hv = _att_mat(att_src_hv, att_dst_hv)
    sv = (scale * (b_lin_v - bn_mean) + bn_beta)[None, :]
    sh = (scale * (b_lin_h - bn_mean) + bn_beta)[None, :]

    co_t = coexistence_data.T
    vh, hht, tabt = _prep(
        virus_data, host_data, coexistence_data, co_t,
        W_gat_v, am_v, W_gat_h, am_h, W_gat_vh, am_vh, W_gat_hv, am_hv,
        W_lin_v, W_lin_h, scale[None, :], sv, sh)

    ar_big = jnp.arange(_NBIG, dtype=jnp.int32)
    ar_co = jnp.arange(_NCO, dtype=jnp.int32)
    ei_v, ei_h, ei_vhv, ei_vhh, src_all, dst_all = _edges(
        virus_edge_index, host_edge_index, coexistence_edge_index,
        coexistence_edge_index_t, ar_big, ar_co)

    alpha_flat = _make_alpha_kernel()(src_all, dst_all, tabt)
    alpha_all = alpha_flat.reshape(2, _H, _ECORE)

    alpha_v = alpha_all[0, :, :_EBIG].T
    alpha_vhv = alpha_all[0, :, _EBIG:].T
    alpha_h = alpha_all[1, :, :_EBIG].T
    alpha_vhh = alpha_all[1, :, _EBIG:].T

    P, Pb, P2 = _bigmm(vh, hht)

    return (P, Pb, P2, (ei_v, alpha_v), (ei_h, alpha_h),
            (ei_vhv, alpha_vhv), (ei_vhh, alpha_vhh))
